# Initial kernel scaffold; baseline (speedup 1.0000x reference)
import jax
import jax.numpy as jnp
from jax.experimental import pallas as pl


def kernel(x, edge_index, edge_attr, batch, global_feats, W_root0, W_nbr0, W_edge0, b0, ln_g0, ln_b0, W_root1, W_nbr1, W_edge1, b1, ln_g1, ln_b1, W_root2, W_nbr2, W_edge2, b2, ln_g2, ln_b2, Wv, bv):
    raise NotImplementedError("write your pallas kernel here")


# speedup vs baseline: 2.6748x; 2.6748x over previous
import jax
import jax.numpy as jnp
from jax import lax
from jax.experimental import pallas as pl
from jax.experimental.pallas import tpu as pltpu
from jax.experimental.pallas import tpu_sc as plsc

NC = 2
NS = 16
NW = NC * NS
CHUNK = 128
ACC_ROWS = 10240
RPT = ACC_ROWS // NS
BM = 2000


def _seg_sum(y, srcp, dstp, zrow):
    cpt = srcp.shape[0] // (NW * CHUNK)
    n, h = y.shape

    mesh = plsc.VectorSubcoreMesh(core_axis_name="c", subcore_axis_name="s")
    out_type = [jax.ShapeDtypeStruct((NC * ACC_ROWS, h), jnp.float32)]
    scratch = [
        pltpu.VMEM((CHUNK,), jnp.int32),
        pltpu.VMEM((CHUNK,), jnp.int32),
        pltpu.VMEM((CHUNK, h), jnp.float32),
        pltpu.VMEM_SHARED((ACC_ROWS, h), jnp.float32),
        pltpu.SemaphoreType.DMA,
    ]

    def body(y_h, srcp_h, dstp_h, z_h, out_h, src_v, dst_v, rows_v, acc, sem):
        cid = lax.axis_index("c")
        sid = lax.axis_index("s")
        wid = cid * NS + sid
        pltpu.sync_copy(z_h, rows_v)
        for r in range(RPT // CHUNK):
            pltpu.sync_copy(rows_v, acc.at[pl.ds(sid * RPT + r * CHUNK, CHUNK)])
        plsc.subcore_barrier()

        def step(j, carry):
            row = wid * cpt + j
            pltpu.sync_copy(srcp_h.at[pl.ds(row * CHUNK, CHUNK)], src_v)
            pltpu.sync_copy(dstp_h.at[pl.ds(row * CHUNK, CHUNK)], dst_v)
            pltpu.async_copy(y_h.at[src_v], rows_v, sem).wait()
            pltpu.sync_copy(rows_v, acc.at[dst_v], add=True)
            return carry

        lax.fori_loop(0, cpt, step, 0)
        plsc.subcore_barrier()
        for r in range(RPT // CHUNK):
            off = sid * RPT + r * CHUNK
            pltpu.sync_copy(acc.at[pl.ds(off, CHUNK)], rows_v)
            pltpu.sync_copy(rows_v, out_h.at[pl.ds(cid * ACC_ROWS + off, CHUNK)])

    fn = pl.kernel(body, mesh=mesh, out_type=out_type, scratch_types=scratch)
    return fn(y, srcp, dstp, zrow)[0]


def _ea_seg_sum(eap, dstp, zrow_e):
    cpt = dstp.shape[0] // (NW * CHUNK)
    de = eap.shape[1]

    mesh = plsc.VectorSubcoreMesh(core_axis_name="c", subcore_axis_name="s")
    out_type = [jax.ShapeDtypeStruct((NC * ACC_ROWS, de), jnp.float32)]
    scratch = [
        pltpu.VMEM((CHUNK,), jnp.int32),
        pltpu.VMEM((CHUNK, de), jnp.float32),
        pltpu.VMEM_SHARED((ACC_ROWS, de), jnp.float32),
    ]

    def body(eap_h, dstp_h, ze_h, out_h, dst_v, ea_v, acc_e):
        cid = lax.axis_index("c")
        sid = lax.axis_index("s")
        wid = cid * NS + sid
        pltpu.sync_copy(ze_h, ea_v)
        for r in range(RPT // CHUNK):
            pltpu.sync_copy(ea_v, acc_e.at[pl.ds(sid * RPT + r * CHUNK, CHUNK)])
        plsc.subcore_barrier()

        def step(j, carry):
            row = wid * cpt + j
            pltpu.sync_copy(dstp_h.at[pl.ds(row * CHUNK, CHUNK)], dst_v)
            pltpu.sync_copy(eap_h.at[pl.ds(row * CHUNK, CHUNK)], ea_v)
            pltpu.sync_copy(ea_v, acc_e.at[dst_v], add=True)
            return carry

        lax.fori_loop(0, cpt, step, 0)
        plsc.subcore_barrier()
        for r in range(RPT // CHUNK):
            off = sid * RPT + r * CHUNK
            pltpu.sync_copy(acc_e.at[pl.ds(off, CHUNK)], ea_v)
            pltpu.sync_copy(ea_v, out_h.at[pl.ds(cid * ACC_ROWS + off, CHUNK)])

    fn = pl.kernel(body, mesh=mesh, out_type=out_type, scratch_types=scratch)
    return fn(eap, dstp, zrow_e)[0]


def _mm_body(x_ref, w_ref, o_ref):
    o_ref[...] = jnp.dot(x_ref[...], w_ref[...],
                         preferred_element_type=jnp.float32)


def _mm(x, w):
    n, d = x.shape
    h = w.shape[1]
    return pl.pallas_call(
        _mm_body,
        grid=(n // BM,),
        in_specs=[pl.BlockSpec((BM, d), lambda i: (i, 0)),
                  pl.BlockSpec((d, h), lambda i: (0, 0))],
        out_specs=pl.BlockSpec((BM, h), lambda i: (i, 0)),
        out_shape=jax.ShapeDtypeStruct((n, h), jnp.float32),
    )(x, w)


def _update_body(h_ref, s_ref, a_ref, wr_ref, we_ref, b_ref, g_ref, be_ref,
                 wn_ref, hout_ref, yout_ref):
    h = h_ref[...]
    t = jnp.dot(h, wr_ref[...], preferred_element_type=jnp.float32)
    t = t + s_ref[0] + s_ref[1]
    t = t + jnp.dot(a_ref[0] + a_ref[1], we_ref[...],
                    preferred_element_type=jnp.float32)
    t = t + b_ref[...]
    mu = jnp.mean(t, axis=-1, keepdims=True)
    var = jnp.mean(jnp.square(t - mu), axis=-1, keepdims=True)
    t = (t - mu) * lax.rsqrt(var + 1e-5) * g_ref[...] + be_ref[...]
    hn = jnp.maximum(t, 0.0) + h
    hout_ref[...] = hn
    yout_ref[...] = jnp.dot(hn, wn_ref[...], preferred_element_type=jnp.float32)


def _update(h, seg, aseg, wr, we, b, g, be, wn):
    n, hd = h.shape
    de = aseg.shape[2]
    specs = [
        pl.BlockSpec((BM, hd), lambda i: (i, 0)),
        pl.BlockSpec((NC, BM, hd), lambda i: (0, i, 0)),
        pl.BlockSpec((NC, BM, de), lambda i: (0, i, 0)),
        pl.BlockSpec((hd, hd), lambda i: (0, 0)),
        pl.BlockSpec((de, hd), lambda i: (0, 0)),
        pl.BlockSpec((1, hd), lambda i: (0, 0)),
        pl.BlockSpec((1, hd), lambda i: (0, 0)),
        pl.BlockSpec((1, hd), lambda i: (0, 0)),
        pl.BlockSpec((hd, hd), lambda i: (0, 0)),
    ]
    return pl.pallas_call(
        _update_body,
        grid=(n // BM,),
        in_specs=specs,
        out_specs=[pl.BlockSpec((BM, hd), lambda i: (i, 0)),
                   pl.BlockSpec((BM, hd), lambda i: (i, 0))],
        out_shape=[jax.ShapeDtypeStruct((n, hd), jnp.float32),
                   jax.ShapeDtypeStruct((n, hd), jnp.float32)],
    )(h, seg, aseg, wr, we, b, g, be, wn)


def _final_body(h_ref, s_ref, a_ref, wr_ref, we_ref, b_ref, g_ref, be_ref,
                wvh_ref, out_ref):
    h = h_ref[...]
    t = jnp.dot(h, wr_ref[...], preferred_element_type=jnp.float32)
    t = t + s_ref[0] + s_ref[1]
    t = t + jnp.dot(a_ref[0] + a_ref[1], we_ref[...],
                    preferred_element_type=jnp.float32)
    t = t + b_ref[...]
    mu = jnp.mean(t, axis=-1, keepdims=True)
    var = jnp.mean(jnp.square(t - mu), axis=-1, keepdims=True)
    t = (t - mu) * lax.rsqrt(var + 1e-5) * g_ref[...] + be_ref[...]
    hn = jnp.maximum(t, 0.0) + h

    @pl.when(pl.program_id(0) == 0)
    def _():
        out_ref[...] = jnp.zeros_like(out_ref)

    out_ref[...] += jnp.sum(hn * wvh_ref[...]).reshape(1, 1)


def _final(h, seg, aseg, wr, we, b, g, be, wvh):
    n, hd = h.shape
    de = aseg.shape[2]
    specs = [
        pl.BlockSpec((BM, hd), lambda i: (i, 0)),
        pl.BlockSpec((NC, BM, hd), lambda i: (0, i, 0)),
        pl.BlockSpec((NC, BM, de), lambda i: (0, i, 0)),
        pl.BlockSpec((hd, hd), lambda i: (0, 0)),
        pl.BlockSpec((de, hd), lambda i: (0, 0)),
        pl.BlockSpec((1, hd), lambda i: (0, 0)),
        pl.BlockSpec((1, hd), lambda i: (0, 0)),
        pl.BlockSpec((1, hd), lambda i: (0, 0)),
        pl.BlockSpec((1, hd), lambda i: (0, 0)),
    ]
    return pl.pallas_call(
        _final_body,
        grid=(n // BM,),
        in_specs=specs,
        out_specs=pl.BlockSpec((1, 1), lambda i: (0, 0)),
        out_shape=jax.ShapeDtypeStruct((1, 1), jnp.float32),
    )(h, seg, aseg, wr, we, b, g, be, wvh)


def kernel(x, edge_index, edge_attr, batch, global_feats,
           W_root0, W_nbr0, W_edge0, b0, ln_g0, ln_b0,
           W_root1, W_nbr1, W_edge1, b1, ln_g1, ln_b1,
           W_root2, W_nbr2, W_edge2, b2, ln_g2, ln_b2,
           Wv, bv):
    n, _ = x.shape
    e = edge_index.shape[1]
    de = edge_attr.shape[1]
    hd = W_root0.shape[1]

    cpt = -(-e // (NW * CHUNK))
    cpt = cpt + (cpt % 2)
    pad = NW * CHUNK * cpt - e
    src = jnp.concatenate([edge_index[0], jnp.zeros((pad,), jnp.int32)])
    dst = jnp.concatenate([edge_index[1], jnp.full((pad,), n, jnp.int32)])
    eap = jnp.concatenate(
        [edge_attr, jnp.zeros((pad, de), jnp.float32)], axis=0)
    zrow = jnp.zeros((CHUNK, hd), jnp.float32)
    zrow_e = jnp.zeros((CHUNK, de), jnp.float32)

    b0r, g0r, be0r = b0.reshape(1, -1), ln_g0.reshape(1, -1), ln_b0.reshape(1, -1)
    b1r, g1r, be1r = b1.reshape(1, -1), ln_g1.reshape(1, -1), ln_b1.reshape(1, -1)
    b2r, g2r, be2r = b2.reshape(1, -1), ln_g2.reshape(1, -1), ln_b2.reshape(1, -1)

    y0 = _mm(x, W_nbr0)
    aseg = _ea_seg_sum(eap, dst, zrow_e).reshape(NC, ACC_ROWS, de)
    seg0 = _seg_sum(y0, src, dst, zrow).reshape(NC, ACC_ROWS, hd)
    h1, y1 = _update(x, seg0, aseg, W_root0, W_edge0, b0r, g0r, be0r, W_nbr1)
    seg1 = _seg_sum(y1, src, dst, zrow).reshape(NC, ACC_ROWS, hd)
    h2, y2 = _update(h1, seg1, aseg, W_root1, W_edge1, b1r, g1r, be1r, W_nbr2)
    seg2 = _seg_sum(y2, src, dst, zrow).reshape(NC, ACC_ROWS, hd)
    val = _final(h2, seg2, aseg, W_root2, W_edge2, b2r, g2r, be2r,
                 Wv[:hd, 0].reshape(1, -1))

    value = (val[0, 0] / jnp.maximum(jnp.float32(n), 1.0)
             + jnp.dot(global_feats[0], Wv[hd:, 0]) + bv[0])
    return value.reshape(1)


# speedup vs baseline: 3.0989x; 1.1586x over previous
import jax
import jax.numpy as jnp
from jax import lax
from jax.experimental import pallas as pl
from jax.experimental.pallas import tpu as pltpu
from jax.experimental.pallas import tpu_sc as plsc

NC = 2
NS = 16
NW = NC * NS
CHUNK = 128
ACC_ROWS = 10240
RPT = ACC_ROWS // NS
BM = 2000


def _seg_sum(y, srcp, dstp, zrow):
    cpt = srcp.shape[0] // (NW * CHUNK)
    n, h = y.shape

    mesh = plsc.VectorSubcoreMesh(core_axis_name="c", subcore_axis_name="s")
    out_type = [jax.ShapeDtypeStruct((NC * ACC_ROWS, h), jnp.float32)]
    scratch = [
        [pltpu.VMEM((CHUNK,), jnp.int32)] * 2,
        [pltpu.VMEM((CHUNK,), jnp.int32)] * 2,
        [pltpu.VMEM((CHUNK, h), jnp.float32)] * 2,
        pltpu.VMEM_SHARED((ACC_ROWS, h), jnp.float32),
        [pltpu.SemaphoreType.DMA] * 2,
    ]

    def body(y_h, srcp_h, dstp_h, z_h, out_h, src_v, dst_v, rows_v, acc, sem):
        cid = lax.axis_index("c")
        sid = lax.axis_index("s")
        wid = cid * NS + sid
        pltpu.sync_copy(z_h, rows_v[0])
        for r in range(RPT // CHUNK):
            pltpu.sync_copy(rows_v[0], acc.at[pl.ds(sid * RPT + r * CHUNK, CHUNK)])
        plsc.subcore_barrier()

        def load_idx(j, b):
            off = (wid * cpt + j) * CHUNK
            pltpu.sync_copy(srcp_h.at[pl.ds(off, CHUNK)], src_v[b])
            pltpu.sync_copy(dstp_h.at[pl.ds(off, CHUNK)], dst_v[b])

        load_idx(0, 0)
        pltpu.async_copy(y_h.at[src_v[0]], rows_v[0], sem[0])

        def step2(jj, carry):
            j = jj * 2
            for b in (0, 1):
                jn = j + b + 1
                bn = (b + 1) % 2

                def fire():
                    load_idx(jn, bn)
                    pltpu.async_copy(y_h.at[src_v[bn]], rows_v[bn], sem[bn])
                if b == 0:
                    fire()
                else:
                    pl.when(jn < cpt)(fire)
                pltpu.make_async_copy(y_h.at[src_v[b]], rows_v[b], sem[b]).wait()
                pltpu.sync_copy(rows_v[b], acc.at[dst_v[b]], add=True)
            return carry

        lax.fori_loop(0, cpt // 2, step2, 0)
        plsc.subcore_barrier()
        for r in range(RPT // CHUNK):
            off = sid * RPT + r * CHUNK
            pltpu.sync_copy(acc.at[pl.ds(off, CHUNK)], rows_v[0])
            pltpu.sync_copy(rows_v[0], out_h.at[pl.ds(cid * ACC_ROWS + off, CHUNK)])

    fn = pl.kernel(body, mesh=mesh, out_type=out_type, scratch_types=scratch)
    return fn(y, srcp, dstp, zrow)[0]


def _ea_seg_sum(eap128, dstp, zrow):
    cpt = dstp.shape[0] // (NW * CHUNK)
    de = 16

    mesh = plsc.VectorSubcoreMesh(core_axis_name="c", subcore_axis_name="s")
    out_type = [jax.ShapeDtypeStruct((NC * ACC_ROWS, CHUNK), jnp.float32)]
    scratch = [
        pltpu.VMEM((CHUNK,), jnp.int32),
        pltpu.VMEM((CHUNK // 8, CHUNK), jnp.float32),
        pltpu.VMEM((CHUNK, CHUNK), jnp.float32),
        pltpu.VMEM_SHARED((ACC_ROWS, CHUNK), jnp.float32),
    ]

    def body(eap_h, dstp_h, z_h, out_h, dst_v, ea_v, rows_v, acc):
        cid = lax.axis_index("c")
        sid = lax.axis_index("s")
        wid = cid * NS + sid
        pltpu.sync_copy(z_h, rows_v)
        for r in range(RPT // CHUNK):
            pltpu.sync_copy(rows_v, acc.at[pl.ds(sid * RPT + r * CHUNK, CHUNK)])
        plsc.subcore_barrier()

        def step(j, carry):
            row = wid * cpt + j
            pltpu.sync_copy(dstp_h.at[pl.ds(row * CHUNK, CHUNK)], dst_v)
            pltpu.sync_copy(eap_h.at[pl.ds(row * (CHUNK // 8), CHUNK // 8)], ea_v)
            for o in range(CHUNK // 8):
                for k in range(8):
                    rows_v[o * 8 + k, pl.ds(0, de)] = ea_v[o, pl.ds(k * de, de)]
            pltpu.sync_copy(rows_v, acc.at[dst_v], add=True)
            return carry

        lax.fori_loop(0, cpt, step, 0)
        plsc.subcore_barrier()
        for r in range(RPT // CHUNK):
            off = sid * RPT + r * CHUNK
            pltpu.sync_copy(acc.at[pl.ds(off, CHUNK)], rows_v)
            pltpu.sync_copy(rows_v, out_h.at[pl.ds(cid * ACC_ROWS + off, CHUNK)])

    fn = pl.kernel(body, mesh=mesh, out_type=out_type, scratch_types=scratch)
    return fn(eap128, dstp, zrow)[0]


def _mm_body(x_ref, w_ref, o_ref):
    o_ref[...] = jnp.dot(x_ref[...], w_ref[...],
                         preferred_element_type=jnp.float32)


def _mm(x, w):
    n, d = x.shape
    h = w.shape[1]
    return pl.pallas_call(
        _mm_body,
        grid=(n // BM,),
        in_specs=[pl.BlockSpec((BM, d), lambda i: (i, 0)),
                  pl.BlockSpec((d, h), lambda i: (0, 0))],
        out_specs=pl.BlockSpec((BM, h), lambda i: (i, 0)),
        out_shape=jax.ShapeDtypeStruct((n, h), jnp.float32),
    )(x, w)


def _update_body(h_ref, s_ref, a_ref, wr_ref, we_ref, b_ref, g_ref, be_ref,
                 wn_ref, hout_ref, yout_ref):
    h = h_ref[...]
    t = jnp.dot(h, wr_ref[...], preferred_element_type=jnp.float32)
    t = t + s_ref[0] + s_ref[1]
    t = t + jnp.dot(a_ref[0] + a_ref[1], we_ref[...],
                    preferred_element_type=jnp.float32)
    t = t + b_ref[...]
    mu = jnp.mean(t, axis=-1, keepdims=True)
    var = jnp.mean(jnp.square(t - mu), axis=-1, keepdims=True)
    t = (t - mu) * lax.rsqrt(var + 1e-5) * g_ref[...] + be_ref[...]
    hn = jnp.maximum(t, 0.0) + h
    hout_ref[...] = hn
    yout_ref[...] = jnp.dot(hn, wn_ref[...], preferred_element_type=jnp.float32)


def _update(h, seg, aseg, wr, we, b, g, be, wn):
    n, hd = h.shape
    de = aseg.shape[2]
    specs = [
        pl.BlockSpec((BM, hd), lambda i: (i, 0)),
        pl.BlockSpec((NC, BM, hd), lambda i: (0, i, 0)),
        pl.BlockSpec((NC, BM, de), lambda i: (0, i, 0)),
        pl.BlockSpec((hd, hd), lambda i: (0, 0)),
        pl.BlockSpec((de, hd), lambda i: (0, 0)),
        pl.BlockSpec((1, hd), lambda i: (0, 0)),
        pl.BlockSpec((1, hd), lambda i: (0, 0)),
        pl.BlockSpec((1, hd), lambda i: (0, 0)),
        pl.BlockSpec((hd, hd), lambda i: (0, 0)),
    ]
    return pl.pallas_call(
        _update_body,
        grid=(n // BM,),
        in_specs=specs,
        out_specs=[pl.BlockSpec((BM, hd), lambda i: (i, 0)),
                   pl.BlockSpec((BM, hd), lambda i: (i, 0))],
        out_shape=[jax.ShapeDtypeStruct((n, hd), jnp.float32),
                   jax.ShapeDtypeStruct((n, hd), jnp.float32)],
    )(h, seg, aseg, wr, we, b, g, be, wn)


def _final_body(h_ref, s_ref, a_ref, wr_ref, we_ref, b_ref, g_ref, be_ref,
                wvh_ref, out_ref):
    h = h_ref[...]
    t = jnp.dot(h, wr_ref[...], preferred_element_type=jnp.float32)
    t = t + s_ref[0] + s_ref[1]
    t = t + jnp.dot(a_ref[0] + a_ref[1], we_ref[...],
                    preferred_element_type=jnp.float32)
    t = t + b_ref[...]
    mu = jnp.mean(t, axis=-1, keepdims=True)
    var = jnp.mean(jnp.square(t - mu), axis=-1, keepdims=True)
    t = (t - mu) * lax.rsqrt(var + 1e-5) * g_ref[...] + be_ref[...]
    hn = jnp.maximum(t, 0.0) + h

    @pl.when(pl.program_id(0) == 0)
    def _():
        out_ref[...] = jnp.zeros_like(out_ref)

    out_ref[...] += jnp.sum(hn * wvh_ref[...]).reshape(1, 1)


def _final(h, seg, aseg, wr, we, b, g, be, wvh):
    n, hd = h.shape
    de = aseg.shape[2]
    specs = [
        pl.BlockSpec((BM, hd), lambda i: (i, 0)),
        pl.BlockSpec((NC, BM, hd), lambda i: (0, i, 0)),
        pl.BlockSpec((NC, BM, de), lambda i: (0, i, 0)),
        pl.BlockSpec((hd, hd), lambda i: (0, 0)),
        pl.BlockSpec((de, hd), lambda i: (0, 0)),
        pl.BlockSpec((1, hd), lambda i: (0, 0)),
        pl.BlockSpec((1, hd), lambda i: (0, 0)),
        pl.BlockSpec((1, hd), lambda i: (0, 0)),
        pl.BlockSpec((1, hd), lambda i: (0, 0)),
    ]
    return pl.pallas_call(
        _final_body,
        grid=(n // BM,),
        in_specs=specs,
        out_specs=pl.BlockSpec((1, 1), lambda i: (0, 0)),
        out_shape=jax.ShapeDtypeStruct((1, 1), jnp.float32),
    )(h, seg, aseg, wr, we, b, g, be, wvh)


def kernel(x, edge_index, edge_attr, batch, global_feats,
           W_root0, W_nbr0, W_edge0, b0, ln_g0, ln_b0,
           W_root1, W_nbr1, W_edge1, b1, ln_g1, ln_b1,
           W_root2, W_nbr2, W_edge2, b2, ln_g2, ln_b2,
           Wv, bv):
    n, _ = x.shape
    e = edge_index.shape[1]
    de = edge_attr.shape[1]
    hd = W_root0.shape[1]

    cpt = -(-e // (NW * CHUNK))
    cpt = cpt + (cpt % 2)
    pad = NW * CHUNK * cpt - e
    src = jnp.concatenate([edge_index[0], jnp.zeros((pad,), jnp.int32)])
    spare = ACC_ROWS - n
    pad_dst = n + jnp.arange(pad, dtype=jnp.int32) % spare
    dst = jnp.concatenate([edge_index[1], pad_dst])
    eap = jnp.concatenate(
        [edge_attr, jnp.zeros((pad, de), jnp.float32)], axis=0)
    zrow = jnp.zeros((CHUNK, hd), jnp.float32)

    b0r, g0r, be0r = b0.reshape(1, -1), ln_g0.reshape(1, -1), ln_b0.reshape(1, -1)
    b1r, g1r, be1r = b1.reshape(1, -1), ln_g1.reshape(1, -1), ln_b1.reshape(1, -1)
    b2r, g2r, be2r = b2.reshape(1, -1), ln_g2.reshape(1, -1), ln_b2.reshape(1, -1)

    y0 = _mm(x, W_nbr0)
    eap128 = eap.reshape(-1, 8 * de)
    aseg = _ea_seg_sum(eap128, dst, zrow)
    aseg = aseg.reshape(NC, ACC_ROWS, CHUNK)[:, :, :de]
    seg0 = _seg_sum(y0, src, dst, zrow).reshape(NC, ACC_ROWS, hd)
    h1, y1 = _update(x, seg0, aseg, W_root0, W_edge0, b0r, g0r, be0r, W_nbr1)
    seg1 = _seg_sum(y1, src, dst, zrow).reshape(NC, ACC_ROWS, hd)
    h2, y2 = _update(h1, seg1, aseg, W_root1, W_edge1, b1r, g1r, be1r, W_nbr2)
    seg2 = _seg_sum(y2, src, dst, zrow).reshape(NC, ACC_ROWS, hd)
    val = _final(h2, seg2, aseg, W_root2, W_edge2, b2r, g2r, be2r,
                 Wv[:hd, 0].reshape(1, -1))

    value = (val[0, 0] / jnp.maximum(jnp.float32(n), 1.0)
             + jnp.dot(global_feats[0], Wv[hd:, 0]) + bv[0])
    return value.reshape(1)
